# baseline (speedup 1.0000x reference)
import functools

import jax
import jax.numpy as jnp
from jax import lax
from jax.experimental import pallas as pl
from jax.experimental.pallas import tpu as pltpu
from jax.experimental.pallas import tpu_sc as plsc

_NC = 2
_NS = 16
_CH = 80
_SEG = 25
_NP = 10240


@functools.lru_cache(maxsize=None)
def _build_sc(N, E, DF, DE):
    R = E // _CH
    K = R // (_NC * _NS)
    assert R % (_NC * _NS) == 0
    stripe = _NP // _NS
    mesh = plsc.VectorSubcoreMesh(core_axis_name="c", subcore_axis_name="s",
                                  num_cores=_NC, num_subcores=_NS)

    @functools.partial(
        pl.kernel,
        out_type=(
            jax.ShapeDtypeStruct((_NC * _NP, DF), jnp.float32),
            jax.ShapeDtypeStruct((_NC * _NP, DE), jnp.float32),
            jax.ShapeDtypeStruct((_NC * _NP, DE), jnp.float32),
        ),
        mesh=mesh,
        compiler_params=pltpu.CompilerParams(use_tc_tiling_on_sc=False),
        scratch_types=[
            pltpu.VMEM_SHARED((_NP, DF), jnp.float32),
            pltpu.VMEM_SHARED((_NP, DE), jnp.float32),
            pltpu.VMEM_SHARED((_NP, DE), jnp.float32),
            pltpu.VMEM((2, _CH), jnp.int32),
            pltpu.VMEM((_CH, DE), jnp.float32),
            pltpu.VMEM((_CH, DE), jnp.float32),
            pltpu.VMEM((_CH, DF), jnp.float32),
            pltpu.SemaphoreType.DMA,
            pltpu.SemaphoreType.DMA,
            pltpu.SemaphoreType.DMA,
        ],
    )
    def sc_k(h_hbm, ei_hbm, ef_hbm, z_f_hbm, z_e_hbm,
             out_h, out_e, out_d,
             acc_h, acc_e, acc_d, sd_v, ef_v, ones_v,
             rows_v, sem, sem2, sem3):
        c = lax.axis_index("c")
        s = lax.axis_index("s")
        wid = s * _NC + c
        sb = s * stripe
        for t in range(stripe // _CH):
            o = sb + t * _CH
            pltpu.sync_copy(z_f_hbm.at[pl.ds(o, _CH)], acc_h.at[pl.ds(o, _CH)])
        pltpu.sync_copy(z_e_hbm.at[pl.ds(sb, stripe)], acc_e.at[pl.ds(sb, stripe)])
        pltpu.sync_copy(z_e_hbm.at[pl.ds(sb, stripe)], acc_d.at[pl.ds(sb, stripe)])
        one16 = jnp.ones((16,), jnp.float32)
        for r in range(_CH):
            ones_v[r] = one16
        plsc.subcore_barrier()

        @pl.loop(0, K)
        def _(k):
            base = (wid * K + k) * _CH
            c_i = pltpu.async_copy(ei_hbm.at[:, pl.ds(base, _CH)], sd_v, sem3)
            c_ef = pltpu.async_copy(ef_hbm.at[pl.ds(base, _CH)], ef_v, sem2)
            c_i.wait()
            c_h = pltpu.async_copy(h_hbm.at[sd_v.at[0]], rows_v, sem)
            c_ef.wait()
            c_h.wait()
            s1 = pltpu.async_copy(rows_v, acc_h.at[sd_v.at[1]], sem3, add=True)
            s2 = pltpu.async_copy(ef_v, acc_e.at[sd_v.at[1]], sem3, add=True)
            s3 = pltpu.async_copy(ones_v, acc_d.at[sd_v.at[1]], sem3, add=True)
            s1.wait()
            s2.wait()
            s3.wait()

        plsc.subcore_barrier()
        ob = c * _NP + sb
        for t in range(stripe // _CH):
            pltpu.sync_copy(acc_h.at[pl.ds(sb + t * _CH, _CH)],
                            out_h.at[pl.ds(ob + t * _CH, _CH)])
        pltpu.sync_copy(acc_e.at[pl.ds(sb, stripe)], out_e.at[pl.ds(ob, stripe)])
        pltpu.sync_copy(acc_d.at[pl.ds(sb, stripe)], out_d.at[pl.ds(ob, stripe)])

    return sc_k


def _tc_body(hs_ref, ah_ref, ae_ref, ad_ref,
             wsT_ref, wn1T_ref, wn2T_ref, o_ref):
    deg = ad_ref[0, :, 0:1] + ad_ref[1, :, 0:1]
    inv = 1.0 / jnp.where(deg == 0.0, 1.0, deg)
    nm = (ah_ref[0] + ah_ref[1]) * inv
    em = (ae_ref[0] + ae_ref[1]) * inv
    z = (jnp.dot(hs_ref[...], wsT_ref[...])
         + jnp.dot(nm, wn1T_ref[...])
         + jnp.dot(em, wn2T_ref[...]))
    z = jnp.maximum(z, 0.0)
    nrm = jnp.sqrt(jnp.sum(z * z, axis=1, keepdims=True))
    o_ref[...] = z / jnp.where(nrm == 0.0, 1.0, nrm)


@functools.lru_cache(maxsize=None)
def _build_tc(N, DF, DE, DO, blk):
    g = N // blk

    def spec(d):
        return pl.BlockSpec((2, blk, d), lambda i: (0, i, 0))

    full = lambda a, b: pl.BlockSpec((a, b), lambda i: (0, 0))
    return pl.pallas_call(
        _tc_body,
        grid=(g,),
        in_specs=[
            pl.BlockSpec((blk, DF), lambda i: (i, 0)),
            spec(DF), spec(DE), spec(DE),
            full(DF, DO), full(DF, DO), full(DE, DO),
        ],
        out_specs=pl.BlockSpec((blk, DO), lambda i: (i, 0)),
        out_shape=jax.ShapeDtypeStruct((N, DO), jnp.float32),
    )


def kernel(h_neigh, h_self, edge_index, edge_features, W_self, W_neigh):
    N, DF = h_neigh.shape
    E = edge_index.shape[1]
    DE = edge_features.shape[1]
    DO = W_self.shape[0]
    ei = edge_index
    z_f = jnp.zeros((_NP, DF), jnp.float32)
    z_e = jnp.zeros((_NP, DE), jnp.float32)
    out_h, out_e, out_d = _build_sc(N, E, DF, DE)(
        h_neigh, ei, edge_features, z_f, z_e)
    wsT = W_self.T
    wn1T = W_neigh[:, :DF].T
    wn2T = W_neigh[:, DF:].T
    return _build_tc(N, DF, DE, DO, 1000)(
        h_self,
        out_h.reshape(2, _NP, DF),
        out_e.reshape(2, _NP, DE),
        out_d.reshape(2, _NP, DE),
        wsT, wn1T, wn2T)


# speedup vs baseline: 1.3100x; 1.3100x over previous
import functools

import jax
import jax.numpy as jnp
from jax import lax
from jax.experimental import pallas as pl
from jax.experimental.pallas import tpu as pltpu
from jax.experimental.pallas import tpu_sc as plsc

_NC = 2
_NS = 16
_CH = 80
_NP = 10240


@functools.lru_cache(maxsize=None)
def _build_sc(N, E, DF, DE):
    R = E // _CH
    K = R // (_NC * _NS)
    assert R % (_NC * _NS) == 0 and K % 2 == 1 and K >= 5
    stripe = _NP // _NS
    mesh = plsc.VectorSubcoreMesh(core_axis_name="c", subcore_axis_name="s",
                                  num_cores=_NC, num_subcores=_NS)

    @functools.partial(
        pl.kernel,
        out_type=(
            jax.ShapeDtypeStruct((_NC * _NP, DF), jnp.float32),
            jax.ShapeDtypeStruct((_NC * _NP, DE), jnp.float32),
            jax.ShapeDtypeStruct((_NC * _NP, DE), jnp.float32),
        ),
        mesh=mesh,
        compiler_params=pltpu.CompilerParams(use_tc_tiling_on_sc=False),
        scratch_types=[
            pltpu.VMEM_SHARED((_NP, DF), jnp.float32),
            pltpu.VMEM_SHARED((_NP, DE), jnp.float32),
            pltpu.VMEM_SHARED((_NP, DE), jnp.float32),
            pltpu.VMEM((2, _CH), jnp.int32),
            pltpu.VMEM((2, _CH), jnp.int32),
            pltpu.VMEM((_CH, DE), jnp.float32),
            pltpu.VMEM((_CH, DE), jnp.float32),
            pltpu.VMEM((_CH, DF), jnp.float32),
            pltpu.VMEM((_CH, DF), jnp.float32),
            pltpu.VMEM((_CH, DE), jnp.float32),
            pltpu.SemaphoreType.DMA,
            pltpu.SemaphoreType.DMA,
            pltpu.SemaphoreType.DMA,
            pltpu.SemaphoreType.DMA,
            pltpu.SemaphoreType.DMA,
            pltpu.SemaphoreType.DMA,
        ],
    )
    def sc_k(h_hbm, ei_hbm, ef_hbm, z_f_hbm, z_e_hbm,
             out_h, out_e, out_d,
             acc_h, acc_e, acc_d, sd0, sd1, ef0, ef1, rows0, rows1,
             ones_v, si0, si1, sg0, sg1, ss0, ss1):
        c = lax.axis_index("c")
        s = lax.axis_index("s")
        wid = s * _NC + c
        sb = s * stripe
        for t in range(stripe // _CH):
            o = sb + t * _CH
            pltpu.sync_copy(z_f_hbm.at[pl.ds(o, _CH)], acc_h.at[pl.ds(o, _CH)])
        pltpu.sync_copy(z_e_hbm.at[pl.ds(sb, stripe)], acc_e.at[pl.ds(sb, stripe)])
        pltpu.sync_copy(z_e_hbm.at[pl.ds(sb, stripe)], acc_d.at[pl.ds(sb, stripe)])
        one16 = jnp.ones((16,), jnp.float32)
        for r in range(_CH):
            ones_v[r] = one16
        plsc.subcore_barrier()

        kbase = wid * K

        def idx_load(k, sd, si):
            pltpu.async_copy(ei_hbm.at[:, pl.ds((kbase + k) * _CH, _CH)], sd, si)

        def wait_idx(sd, si):
            pltpu.make_async_copy(ei_hbm.at[:, pl.ds(0, _CH)], sd, si).wait()

        def gathers(k, sd, rows, ef, sg):
            pltpu.async_copy(h_hbm.at[sd.at[0]], rows, sg)
            pltpu.async_copy(ef_hbm.at[pl.ds((kbase + k) * _CH, _CH)], ef, sg)

        def wait_gathers(rows, ef, sg):
            pltpu.make_async_copy(h_hbm.at[pl.ds(0, _CH)], rows, sg).wait()
            pltpu.make_async_copy(ef_hbm.at[pl.ds(0, _CH)], ef, sg).wait()

        def scatters(sd, rows, ef, ss):
            pltpu.async_copy(rows, acc_h.at[sd.at[1]], ss, add=True)
            pltpu.async_copy(ef, acc_e.at[sd.at[1]], ss, add=True)
            pltpu.async_copy(ones_v, acc_d.at[sd.at[1]], ss, add=True)

        def wait_scatters(rows, ef, ss):
            pltpu.make_async_copy(rows, acc_h.at[pl.ds(0, _CH)], ss).wait()
            pltpu.make_async_copy(ef, acc_e.at[pl.ds(0, _CH)], ss).wait()
            pltpu.make_async_copy(ones_v, acc_d.at[pl.ds(0, _CH)], ss).wait()

        idx_load(0, sd0, si0)
        idx_load(1, sd1, si1)
        wait_idx(sd0, si0)
        gathers(0, sd0, rows0, ef0, sg0)
        wait_gathers(rows0, ef0, sg0)
        scatters(sd0, rows0, ef0, ss0)
        wait_idx(sd1, si1)
        gathers(1, sd1, rows1, ef1, sg1)

        @pl.loop(0, (K - 3) // 2)
        def _(j):
            k2 = 2 * j + 2
            k3 = 2 * j + 3
            wait_scatters(rows0, ef0, ss0)
            idx_load(k2, sd0, si0)
            wait_gathers(rows1, ef1, sg1)
            scatters(sd1, rows1, ef1, ss1)
            wait_idx(sd0, si0)
            gathers(k2, sd0, rows0, ef0, sg0)
            wait_scatters(rows1, ef1, ss1)
            idx_load(k3, sd1, si1)
            wait_gathers(rows0, ef0, sg0)
            scatters(sd0, rows0, ef0, ss0)
            wait_idx(sd1, si1)
            gathers(k3, sd1, rows1, ef1, sg1)

        wait_scatters(rows0, ef0, ss0)
        idx_load(K - 1, sd0, si0)
        wait_gathers(rows1, ef1, sg1)
        scatters(sd1, rows1, ef1, ss1)
        wait_idx(sd0, si0)
        gathers(K - 1, sd0, rows0, ef0, sg0)
        wait_gathers(rows0, ef0, sg0)
        scatters(sd0, rows0, ef0, ss0)
        wait_scatters(rows1, ef1, ss1)
        wait_scatters(rows0, ef0, ss0)

        plsc.subcore_barrier()
        ob = c * _NP + sb
        for t in range(stripe // _CH):
            pltpu.sync_copy(acc_h.at[pl.ds(sb + t * _CH, _CH)],
                            out_h.at[pl.ds(ob + t * _CH, _CH)])
        pltpu.sync_copy(acc_e.at[pl.ds(sb, stripe)], out_e.at[pl.ds(ob, stripe)])
        pltpu.sync_copy(acc_d.at[pl.ds(sb, stripe)], out_d.at[pl.ds(ob, stripe)])

    return sc_k


def _tc_body(hs_ref, ah_ref, ae_ref, ad_ref,
             wsT_ref, wn1T_ref, wn2T_ref, o_ref):
    deg = ad_ref[0, :, 0:1] + ad_ref[1, :, 0:1]
    inv = 1.0 / jnp.where(deg == 0.0, 1.0, deg)
    nm = (ah_ref[0] + ah_ref[1]) * inv
    em = (ae_ref[0] + ae_ref[1]) * inv
    z = (jnp.dot(hs_ref[...], wsT_ref[...])
         + jnp.dot(nm, wn1T_ref[...])
         + jnp.dot(em, wn2T_ref[...]))
    z = jnp.maximum(z, 0.0)
    nrm = jnp.sqrt(jnp.sum(z * z, axis=1, keepdims=True))
    o_ref[...] = z / jnp.where(nrm == 0.0, 1.0, nrm)


@functools.lru_cache(maxsize=None)
def _build_tc(N, DF, DE, DO, blk):
    g = N // blk

    def spec(d):
        return pl.BlockSpec((2, blk, d), lambda i: (0, i, 0))

    full = lambda a, b: pl.BlockSpec((a, b), lambda i: (0, 0))
    return pl.pallas_call(
        _tc_body,
        grid=(g,),
        in_specs=[
            pl.BlockSpec((blk, DF), lambda i: (i, 0)),
            spec(DF), spec(DE), spec(DE),
            full(DF, DO), full(DF, DO), full(DE, DO),
        ],
        out_specs=pl.BlockSpec((blk, DO), lambda i: (i, 0)),
        out_shape=jax.ShapeDtypeStruct((N, DO), jnp.float32),
    )


def kernel(h_neigh, h_self, edge_index, edge_features, W_self, W_neigh):
    N, DF = h_neigh.shape
    E = edge_index.shape[1]
    DE = edge_features.shape[1]
    DO = W_self.shape[0]
    z_f = jnp.zeros((_NP, DF), jnp.float32)
    z_e = jnp.zeros((_NP, DE), jnp.float32)
    out_h, out_e, out_d = _build_sc(N, E, DF, DE)(
        h_neigh, edge_index, edge_features, z_f, z_e)
    wsT = W_self.T
    wn1T = W_neigh[:, :DF].T
    wn2T = W_neigh[:, DF:].T
    return _build_tc(N, DF, DE, DO, 1000)(
        h_self,
        out_h.reshape(2, _NP, DF),
        out_e.reshape(2, _NP, DE),
        out_d.reshape(2, _NP, DE),
        wsT, wn1T, wn2T)
